# Initial kernel scaffold; baseline (speedup 1.0000x reference)
import jax
import jax.numpy as jnp
from jax.experimental import pallas as pl


def kernel(z, pos_edge_index, neg_edge_index, time_vector_flag, pos_time_vector, neg_time_vector):
    raise NotImplementedError("write your pallas kernel here")


# speedup vs baseline: 3.4604x; 3.4604x over previous
import functools

import jax
import jax.numpy as jnp
from jax import lax
from jax.experimental import pallas as pl
from jax.experimental.pallas import tpu as pltpu
from jax.experimental.pallas import tpu_sc as plsc

_EPS = 1e-15
_L = 16

_GATHER_DNUMS = lax.GatherDimensionNumbers(
    offset_dims=(), collapsed_slice_dims=(0,), start_index_map=(0,))


def _lane_perm(v, idx):
    return lax.gather(v, idx[:, None], dimension_numbers=_GATHER_DNUMS,
                      slice_sizes=(1,),
                      mode=lax.GatherScatterMode.PROMISE_IN_BOUNDS)


def _make_sc_decode(N, E, D):
    NC, NS = 2, 16
    NW = NC * NS
    per_w = E // NW
    B = 80
    n_chunks = per_w // B
    assert per_w % B == 0 and D % _L == 0

    mesh = plsc.VectorSubcoreMesh(core_axis_name="c", subcore_axis_name="s")

    @functools.partial(
        pl.kernel,
        mesh=mesh,
        out_type=(jax.ShapeDtypeStruct((E,), jnp.float32),
                  jax.ShapeDtypeStruct((E,), jnp.float32)),
        scratch_types=[
            pltpu.VMEM((B,), jnp.int32),
            pltpu.VMEM((B,), jnp.int32),
            pltpu.VMEM((B, D), jnp.float32),
            pltpu.VMEM((B, D), jnp.float32),
            pltpu.VMEM((B, D), jnp.float32),
            pltpu.VMEM((B,), jnp.float32),
            pltpu.VMEM((_L,), jnp.int32),
            pltpu.SemaphoreType.DMA,
            pltpu.SemaphoreType.DMA,
        ],
    )
    def sc_decode(z_hbm, pos_a, pos_b, neg_a, neg_b, flag_hbm, pos_t, neg_t,
                  out_pos, out_neg, idx_a, idx_b, srcb, dstb, tb, vb, flagv,
                  sem_a, sem_b):
        wid = lax.axis_index("s") * NC + lax.axis_index("c")
        pltpu.sync_copy(flag_hbm, flagv)
        fv = flagv[...]
        one = jnp.ones((_L,), jnp.float32)
        zero = jnp.zeros((_L,), jnp.float32)
        wB = jnp.where(fv == 2, one, zero)
        wC = jnp.where((fv == 1) | (fv == 2), one, zero)
        lane = lax.iota(jnp.int32, _L)

        for ea, eb, tv, outv in ((pos_a, pos_b, pos_t, out_pos),
                                 (neg_a, neg_b, neg_t, out_neg)):
            def chunk_body(c, _, ea=ea, eb=eb, tv=tv, outv=outv):
                base = pl.multiple_of(wid * per_w + c * B, 8)
                pltpu.sync_copy(ea.at[pl.ds(base, B)], idx_a)
                pltpu.sync_copy(eb.at[pl.ds(base, B)], idx_b)
                cp_a = pltpu.async_copy(z_hbm.at[idx_a], srcb, sem_a)
                cp_b = pltpu.async_copy(z_hbm.at[idx_b], dstb, sem_b)
                pltpu.sync_copy(tv.at[pl.ds(base, B), :], tb)
                cp_a.wait()
                cp_b.wait()

                def group_body(g, _):
                    gbase = pl.multiple_of(g * _L, 8)
                    accs = []
                    for e16 in range(_L):
                        e = gbase + e16
                        acc = zero
                        for k in range(D // _L):
                            s = srcb[e, pl.ds(k * _L, _L)]
                            d = dstb[e, pl.ds(k * _L, _L)]
                            t = tb[e, pl.ds(k * _L, _L)]
                            acc = acc + s * d + (s + d) * (t * wB) \
                                + (t * t) * wC
                        accs.append(acc)
                    for dd in (1, 2, 4, 8):
                        idx = lane ^ dd
                        msk = (lane & dd) == 0
                        accs = [
                            jnp.where(msk,
                                      accs[i] + _lane_perm(accs[i], idx),
                                      accs[i + 1] + _lane_perm(accs[i + 1], idx))
                            for i in range(0, len(accs), 2)
                        ]
                    vb[pl.ds(gbase, _L)] = accs[0]
                    return 0

                lax.fori_loop(0, B // _L, group_body, 0)
                pltpu.sync_copy(vb, outv.at[pl.ds(base, B)])
                return 0

            lax.fori_loop(0, n_chunks, chunk_body, 0)

    return sc_decode


def _loss_body(pv_ref, nv_ref, out_ref, *, n_edges):
    pv = pv_ref[...]
    nv = nv_ref[...]

    def sig(v):
        return jnp.where(v >= 0.0,
                         1.0 / (1.0 + jnp.exp(-v)),
                         jnp.exp(v) / (1.0 + jnp.exp(v)))

    pos_terms = -jnp.log(sig(pv) + _EPS)
    neg_terms = -jnp.log(1.0 - sig(nv) + _EPS)
    inv = jnp.float32(1.0 / n_edges)
    tot = jnp.sum(pos_terms) * inv + jnp.sum(neg_terms) * inv
    out_ref[...] = tot.reshape(1, 1)


def kernel(z, pos_edge_index, neg_edge_index, time_vector_flag,
           pos_time_vector, neg_time_vector):
    N, D = z.shape
    E = pos_edge_index.shape[1]
    flag_arr = jnp.full((_L,), time_vector_flag, dtype=jnp.int32)

    sc_decode = _make_sc_decode(N, E, D)
    v_pos, v_neg = sc_decode(z,
                             pos_edge_index[0], pos_edge_index[1],
                             neg_edge_index[0], neg_edge_index[1],
                             flag_arr, pos_time_vector, neg_time_vector)

    rows = E // 128
    pv = v_pos.reshape(rows, 128)
    nv = v_neg.reshape(rows, 128)
    out = pl.pallas_call(
        functools.partial(_loss_body, n_edges=E),
        out_shape=jax.ShapeDtypeStruct((1, 1), jnp.float32),
    )(pv, nv)
    return out[0, 0]


# speedup vs baseline: 6.8022x; 1.9657x over previous
import functools

import jax
import jax.numpy as jnp
from jax import lax
from jax.experimental import pallas as pl
from jax.experimental.pallas import tpu as pltpu
from jax.experimental.pallas import tpu_sc as plsc

_EPS = 1e-15
_L = 16

_GATHER_DNUMS = lax.GatherDimensionNumbers(
    offset_dims=(), collapsed_slice_dims=(0,), start_index_map=(0,))


def _lane_perm(v, idx):
    return lax.gather(v, idx[:, None], dimension_numbers=_GATHER_DNUMS,
                      slice_sizes=(1,),
                      mode=lax.GatherScatterMode.PROMISE_IN_BOUNDS)


def _make_sc_decode(N, E, D):
    NC, NS = 2, 16
    NW = NC * NS
    per_w = E // NW
    B = 80
    n_chunks = per_w // B
    n_pairs = (n_chunks + 1) // 2
    assert per_w % B == 0 and B % _L == 0 and D % _L == 0

    mesh = plsc.VectorSubcoreMesh(core_axis_name="c", subcore_axis_name="s")

    @functools.partial(
        pl.kernel,
        mesh=mesh,
        out_type=(jax.ShapeDtypeStruct((E,), jnp.float32),
                  jax.ShapeDtypeStruct((E,), jnp.float32)),
        scratch_types=[
            pltpu.VMEM((per_w,), jnp.int32),
            pltpu.VMEM((per_w,), jnp.int32),
            pltpu.VMEM((B, D), jnp.float32),
            pltpu.VMEM((B, D), jnp.float32),
            pltpu.VMEM((B, D), jnp.float32),
            pltpu.VMEM((B, D), jnp.float32),
            pltpu.VMEM((B, D), jnp.float32),
            pltpu.VMEM((B, D), jnp.float32),
            pltpu.VMEM((per_w,), jnp.float32),
            pltpu.VMEM((_L,), jnp.int32),
            pltpu.SemaphoreType.DMA,
            pltpu.SemaphoreType.DMA,
            pltpu.SemaphoreType.DMA,
            pltpu.SemaphoreType.DMA,
            pltpu.SemaphoreType.DMA,
            pltpu.SemaphoreType.DMA,
        ],
    )
    def sc_decode(z_hbm, pos_a, pos_b, neg_a, neg_b, flag_hbm, pos_t, neg_t,
                  out_pos, out_neg, idx_a, idx_b, src0, src1, dst0, dst1,
                  tb0, tb1, vall, flagv,
                  sem_s0, sem_s1, sem_d0, sem_d1, sem_t0, sem_t1):
        wid = lax.axis_index("s") * NC + lax.axis_index("c")
        base_w = pl.multiple_of(wid * per_w, 8)
        pltpu.sync_copy(flag_hbm, flagv)
        fv = flagv[...]
        one = jnp.ones((_L,), jnp.float32)
        zero = jnp.zeros((_L,), jnp.float32)
        wB = jnp.where(fv == 2, one, zero)
        wC = jnp.where((fv == 1) | (fv == 2), one, zero)
        lane = lax.iota(jnp.int32, _L)

        bufsets = ((src0, dst0, tb0, sem_s0, sem_d0, sem_t0),
                   (src1, dst1, tb1, sem_s1, sem_d1, sem_t1))

        for ea, eb, tv, outv in ((pos_a, pos_b, pos_t, out_pos),
                                 (neg_a, neg_b, neg_t, out_neg)):
            pltpu.sync_copy(ea.at[pl.ds(base_w, per_w)], idx_a)
            pltpu.sync_copy(eb.at[pl.ds(base_w, per_w)], idx_b)

            def issue(cc, sb, db, tbuf, ss, sd, st, tv=tv):
                off = pl.multiple_of(cc * B, 8)
                pltpu.async_copy(z_hbm.at[idx_a.at[pl.ds(off, B)]], sb, ss)
                pltpu.async_copy(z_hbm.at[idx_b.at[pl.ds(off, B)]], db, sd)
                pltpu.async_copy(tv.at[pl.ds(base_w + off, B), :], tbuf, st)

            def wait_set(sb, db, tbuf, ss, sd, st, tv=tv):
                pltpu.make_async_copy(
                    z_hbm.at[idx_a.at[pl.ds(0, B)]], sb, ss).wait()
                pltpu.make_async_copy(
                    z_hbm.at[idx_b.at[pl.ds(0, B)]], db, sd).wait()
                pltpu.make_async_copy(
                    tv.at[pl.ds(base_w, B), :], tbuf, st).wait()

            def compute(cc, sb, db, tbuf):
                def group_body(g, _):
                    gbase = pl.multiple_of(g * _L, 8)
                    accs = []
                    for e16 in range(_L):
                        e = gbase + e16
                        acc = zero
                        for k in range(D // _L):
                            s = sb[e, pl.ds(k * _L, _L)]
                            d = db[e, pl.ds(k * _L, _L)]
                            t = tbuf[e, pl.ds(k * _L, _L)]
                            acc = acc + s * d + (s + d) * (t * wB) \
                                + (t * t) * wC
                        accs.append(acc)
                    for dd in (1, 2, 4, 8):
                        idx = lane ^ dd
                        msk = (lane & dd) == 0
                        accs = [
                            jnp.where(
                                msk,
                                accs[i] + _lane_perm(accs[i], idx),
                                accs[i + 1] + _lane_perm(accs[i + 1], idx))
                            for i in range(0, len(accs), 2)
                        ]
                    vall[pl.ds(pl.multiple_of(cc * B + g * _L, 8), _L)] = \
                        accs[0]
                    return 0

                lax.fori_loop(0, B // _L, group_body, 0)

            issue(0, *bufsets[0])

            def pair_body(g, _):
                for ph in (0, 1):
                    cc = 2 * g + ph
                    cur = bufsets[ph]
                    nxt = bufsets[1 - ph]

                    def do_chunk(cc=cc, cur=cur, nxt=nxt):
                        wait_set(*cur)

                        @pl.when(cc + 1 < n_chunks)
                        def _():
                            issue(cc + 1, *nxt)

                        compute(cc, cur[0], cur[1], cur[2])

                    if ph == 0:
                        do_chunk()
                    else:
                        pl.when(cc < n_chunks)(do_chunk)
                return 0

            lax.fori_loop(0, n_pairs, pair_body, 0)
            pltpu.sync_copy(vall, outv.at[pl.ds(base_w, per_w)])

    return sc_decode


def _loss_body(pv_ref, nv_ref, out_ref, *, n_edges):
    pv = pv_ref[...]
    nv = nv_ref[...]

    def sig(v):
        return jnp.where(v >= 0.0,
                         1.0 / (1.0 + jnp.exp(-v)),
                         jnp.exp(v) / (1.0 + jnp.exp(v)))

    pos_terms = -jnp.log(sig(pv) + _EPS)
    neg_terms = -jnp.log(1.0 - sig(nv) + _EPS)
    inv = jnp.float32(1.0 / n_edges)
    tot = jnp.sum(pos_terms) * inv + jnp.sum(neg_terms) * inv
    out_ref[...] = tot.reshape(1, 1)


def kernel(z, pos_edge_index, neg_edge_index, time_vector_flag,
           pos_time_vector, neg_time_vector):
    N, D = z.shape
    E = pos_edge_index.shape[1]
    flag_arr = jnp.full((_L,), time_vector_flag, dtype=jnp.int32)

    sc_decode = _make_sc_decode(N, E, D)
    v_pos, v_neg = sc_decode(z,
                             pos_edge_index[0], pos_edge_index[1],
                             neg_edge_index[0], neg_edge_index[1],
                             flag_arr, pos_time_vector, neg_time_vector)

    rows = E // 128
    pv = v_pos.reshape(rows, 128)
    nv = v_neg.reshape(rows, 128)
    out = pl.pallas_call(
        functools.partial(_loss_body, n_edges=E),
        out_shape=jax.ShapeDtypeStruct((1, 1), jnp.float32),
    )(pv, nv)
    return out[0, 0]
